# Initial kernel scaffold; baseline (speedup 1.0000x reference)
import jax
import jax.numpy as jnp
from jax.experimental import pallas as pl


def kernel(x, edge_index, batch, edge_index2, W1a, b1a, W1b, b1b, We, be, W21, b21, W22, b22, W23, b23, W24, b24, Ws1, bs1, Ws2, bs2, Wr):
    raise NotImplementedError("write your pallas kernel here")


# speedup vs baseline: 2.9120x; 2.9120x over previous
import functools

import jax
import jax.numpy as jnp
from jax import lax
from jax.experimental import pallas as pl
from jax.experimental.pallas import tpu as pltpu
from jax.experimental.pallas import tpu_sc as plsc

N = 10000
F_IN = 128
F = 256
H = 128
G = 512
E2 = 8192

CHUNK = 128
CPT = 157
TILE_E = CHUNK * CPT
EPAD = 16 * TILE_E
NP = N // 2
ACC_ROWS = 16 * 320
WB = 312

RB = 400
NB = N // RB


def _sc_agg_body(m_hbm, src_hbm, dst_hbm, zeros_hbm, out_hbm,
                 fidx_t, dsta_t, dstb_t, rows_v, acc, sem):
    cid = lax.axis_index("c")
    sid = lax.axis_index("s")

    pltpu.sync_copy(src_hbm.at[sid], fidx_t)
    pltpu.sync_copy(dst_hbm.at[sid], dsta_t)
    col0 = cid * N

    def prep(j, carry2):
        c = j // (CHUNK // 16)
        k = (j % (CHUNK // 16)) * 16
        d = dsta_t[c, pl.ds(k, 16)]
        da = jnp.where(d < NP, d, NP)
        db = d - NP
        db = jnp.where(db >= 0, db, NP)
        dsta_t[c, pl.ds(k, 16)] = da
        dstb_t[c, pl.ds(k, 16)] = db
        fidx_t[c, pl.ds(k, 16)] = fidx_t[c, pl.ds(k, 16)] + col0
        return carry2
    lax.fori_loop(0, CPT * (CHUNK // 16), prep, 0)

    for p, dst_t in ((0, dsta_t), (1, dstb_t)):
        pltpu.sync_copy(zeros_hbm.at[pl.ds(sid * 320, 320)],
                        acc.at[pl.ds(sid * 320, 320)])
        plsc.subcore_barrier()

        def chunk_body(c, carry):
            pltpu.async_copy(m_hbm.at[fidx_t.at[c]], rows_v, sem).wait()
            pltpu.sync_copy(rows_v, acc.at[dst_t.at[c]], add=True)
            return carry

        lax.fori_loop(0, CPT, chunk_body, 0)
        plsc.subcore_barrier()

        out0 = col0 + p * NP

        @pl.when(sid < 15)
        def _():
            pltpu.sync_copy(acc.at[pl.ds(sid * WB, WB)],
                            out_hbm.at[pl.ds(out0 + sid * WB, WB)])

        @pl.when(sid == 15)
        def _():
            pltpu.sync_copy(acc.at[pl.ds(15 * WB, NP - 15 * WB)],
                            out_hbm.at[pl.ds(out0 + 15 * WB, NP - 15 * WB)])
        plsc.subcore_barrier()


def _sc_agg(m_stack, src2d, dst2d, zeros):
    kern = pl.kernel(
        _sc_agg_body,
        out_type=jax.ShapeDtypeStruct((2 * N, H), jnp.float32),
        mesh=plsc.VectorSubcoreMesh(core_axis_name="c", subcore_axis_name="s"),
        scratch_types=[
            pltpu.VMEM((CPT, CHUNK), jnp.int32),
            pltpu.VMEM((CPT, CHUNK), jnp.int32),
            pltpu.VMEM((CPT, CHUNK), jnp.int32),
            pltpu.VMEM((CHUNK, H), jnp.float32),
            pltpu.VMEM_SHARED((ACC_ROWS, H), jnp.float32),
            pltpu.SemaphoreType.DMA,
        ],
    )
    return kern(m_stack, src2d, dst2d, zeros)


def _mm1_body(x_ref, w_ref, o_ref):
    o_ref[0] = jnp.dot(x_ref[...], w_ref[...], preferred_element_type=jnp.float32)


def _mm1(x, W1a):
    return pl.pallas_call(
        _mm1_body,
        grid=(NB, 2),
        in_specs=[
            pl.BlockSpec((RB, F_IN), lambda i, h: (i, 0)),
            pl.BlockSpec((F_IN, H), lambda i, h: (0, h)),
        ],
        out_specs=pl.BlockSpec((1, RB, H), lambda i, h: (h, i, 0)),
        out_shape=jax.ShapeDtypeStruct((2, N, H), jnp.float32),
    )(x, W1a)


def _combine_mm_body(a0_ref, a1_ref, b_ref, w_ref, o_ref):
    agg = jnp.concatenate([a0_ref[0], a1_ref[0]], axis=1)
    h = jnp.maximum(agg + b_ref[...], 0.0)
    o_ref[0] = jnp.dot(h, w_ref[...], preferred_element_type=jnp.float32)


def _combine_mm(agg3, b, W):
    return pl.pallas_call(
        _combine_mm_body,
        grid=(NB, 2),
        in_specs=[
            pl.BlockSpec((1, RB, H), lambda i, h: (0, i, 0)),
            pl.BlockSpec((1, RB, H), lambda i, h: (1, i, 0)),
            pl.BlockSpec((1, F), lambda i, h: (0, 0)),
            pl.BlockSpec((F, H), lambda i, h: (0, h)),
        ],
        out_specs=pl.BlockSpec((1, RB, H), lambda i, h: (h, i, 0)),
        out_shape=jax.ShapeDtypeStruct((2, N, H), jnp.float32),
    )(agg3, agg3, b.reshape(1, F), W)


def _pool_body(a0_ref, a1_ref, b_ref, batch_ref, o_ref, acc, cnt):
    i = pl.program_id(0)

    @pl.when(i == 0)
    def _():
        acc[...] = jnp.zeros_like(acc)
        cnt[...] = jnp.zeros_like(cnt)

    agg = jnp.concatenate([a0_ref[0], a1_ref[0]], axis=1)
    h = jnp.maximum(agg + b_ref[...], 0.0)
    seg = batch_ref[0, 0]
    iot = lax.broadcasted_iota(jnp.int32, (G, RB), 0)
    P = (iot == seg[None, :]).astype(jnp.float32)
    acc[...] += jnp.dot(P, h, preferred_element_type=jnp.float32)
    cnt[...] += jnp.dot(P, jnp.ones((RB, H), jnp.float32),
                        preferred_element_type=jnp.float32)

    @pl.when(i == pl.num_programs(0) - 1)
    def _():
        c = jnp.maximum(cnt[...], 1.0)
        o_ref[...] = acc[...] / jnp.concatenate([c, c], axis=1)


def _pool(agg3, b, batch3):
    return pl.pallas_call(
        _pool_body,
        grid=(NB,),
        in_specs=[
            pl.BlockSpec((1, RB, H), lambda i: (0, i, 0)),
            pl.BlockSpec((1, RB, H), lambda i: (1, i, 0)),
            pl.BlockSpec((1, F), lambda i: (0, 0)),
            pl.BlockSpec((1, 1, RB), lambda i: (i, 0, 0)),
        ],
        out_specs=pl.BlockSpec((G, F), lambda i: (0, 0)),
        out_shape=jax.ShapeDtypeStruct((G, F), jnp.float32),
        scratch_shapes=[
            pltpu.VMEM((G, F), jnp.float32),
            pltpu.VMEM((G, H), jnp.float32),
        ],
    )(agg3, agg3, b.reshape(1, F), batch3)


def _smallnet_body(hg_ref, src_ref, dst_ref,
                   we_ref, be_ref, w21_ref, b21_ref, w22_ref, b22_ref,
                   w23_ref, b23_ref, w24_ref, b24_ref,
                   ws1_ref, bs1_ref, ws2_ref, bs2_ref, wr_ref,
                   of_ref, lg_ref, sg_ref):
    iot = lax.broadcasted_iota(jnp.int32, (G, 1024), 0)
    a2 = jnp.zeros((G, G), jnp.float32)
    for k in range(E2 // 1024):
        s = src_ref[0, pl.ds(k * 1024, 1024)]
        d = dst_ref[0, pl.ds(k * 1024, 1024)]
        Dk = (iot == d[None, :]).astype(jnp.bfloat16)
        Sk = (iot == s[None, :]).astype(jnp.bfloat16)
        a2 = a2 + lax.dot_general(Dk, Sk, (((1,), (1,)), ((), ())),
                                  preferred_element_type=jnp.float32)

    def gcn(h, w_ref2, b_ref2):
        m = jnp.dot(h, w_ref2[...], preferred_element_type=jnp.float32)
        agg = jnp.dot(a2, m, preferred_element_type=jnp.float32)
        return jnp.maximum(agg + b_ref2[...], 0.0)

    h = jnp.dot(hg_ref[...], we_ref[...], preferred_element_type=jnp.float32)
    h = h + be_ref[...]
    feats = [h]
    for w_r, b_r in ((w21_ref, b21_ref), (w22_ref, b22_ref),
                     (w23_ref, b23_ref), (w24_ref, b24_ref)):
        h = gcn(h, w_r, b_r)
        feats.append(h)
    sc1 = gcn(feats[2], ws1_ref, bs1_ref)
    sc2 = gcn(feats[4], ws2_ref, bs2_ref)
    out_feat = jnp.concatenate([sc1, sc2], axis=1)
    logits = jnp.sum(out_feat * wr_ref[...], axis=1)
    of_ref[...] = out_feat
    lg_ref[...] = logits[None, :]
    sg_ref[...] = (1.0 / (1.0 + jnp.exp(-logits)))[None, :]


def _smallnet(hg, src2, dst2, We, be, W21, b21, W22, b22, W23, b23, W24, b24,
              Ws1, bs1, Ws2, bs2, Wr):
    full = lambda shape: pl.BlockSpec(shape, lambda: tuple(0 for _ in shape))
    args = [hg, src2, dst2,
            We, be.reshape(1, F), W21, b21.reshape(1, G), W22, b22.reshape(1, G),
            W23, b23.reshape(1, 1024), W24, b24.reshape(1, 1024),
            Ws1, bs1.reshape(1, H), Ws2, bs2.reshape(1, H), Wr.reshape(1, F)]
    return pl.pallas_call(
        _smallnet_body,
        in_specs=[full(a.shape) for a in args],
        out_specs=[full((G, F)), full((1, G)), full((1, G))],
        out_shape=[
            jax.ShapeDtypeStruct((G, F), jnp.float32),
            jax.ShapeDtypeStruct((1, G), jnp.float32),
            jax.ShapeDtypeStruct((1, G), jnp.float32),
        ],
    )(*args)


def kernel(x, edge_index, batch, edge_index2, W1a, b1a, W1b, b1b, We, be,
           W21, b21, W22, b22, W23, b23, W24, b24, Ws1, bs1, Ws2, bs2, Wr):
    src = jnp.concatenate(
        [edge_index[0], jnp.zeros((EPAD - edge_index.shape[1],), jnp.int32)])
    dst = jnp.concatenate(
        [edge_index[1],
         jnp.full((EPAD - edge_index.shape[1],), N, jnp.int32)])
    src2d = src.reshape(16, CPT, CHUNK)
    dst2d = dst.reshape(16, CPT, CHUNK)
    zeros = jnp.zeros((ACC_ROWS, H), jnp.float32)

    m1 = _mm1(x, W1a).reshape(2 * N, H)
    agg1 = _sc_agg(m1, src2d, dst2d, zeros).reshape(2, N, H)
    m2 = _combine_mm(agg1, b1a, W1b).reshape(2 * N, H)
    agg2 = _sc_agg(m2, src2d, dst2d, zeros).reshape(2, N, H)

    batch3 = batch.reshape(NB, 1, RB)
    hg = _pool(agg2, b1b, batch3)

    out_feat, lg, sg = _smallnet(
        hg, edge_index2[0].reshape(1, E2), edge_index2[1].reshape(1, E2),
        We, be, W21, b21, W22, b22, W23, b23, W24, b24,
        Ws1, bs1, Ws2, bs2, Wr)
    return (out_feat, lg.reshape(-1), sg.reshape(-1))
